# baseline (speedup 1.0000x reference)
import functools

import jax
import jax.numpy as jnp
import numpy as np
from jax.experimental import pallas as pl
from jax.experimental.pallas import tpu as pltpu

B = 20; NPG = 500; N = B * NPG
EPG = 8000; E = B * EPG
H = 200; T = 5; F = 40
KS = [3, 10, 30]
ESM = 1280; EVO = 1024
NUM_CLASS = 384
DEG_HIST = np.zeros(33); DEG_HIST[16] = N
_bins = np.arange(33).astype(np.float64)
AVG_LIN = float((_bins * DEG_HIST).sum() / DEG_HIST.sum())
AVG_LOG = float((np.log(_bins + 1.0) * DEG_HIST).sum() / DEG_HIST.sum())


def _encoder_body(esm_ref, evo_ref, ws_ref, bs_ref, we_ref, be_ref,
                  w1_ref, b1_ref, out_ref):
    r0 = jax.nn.relu(
        jnp.dot(evo_ref[...], we_ref[...],
                preferred_element_type=jnp.float32) + be_ref[...])
    r1 = jax.nn.relu(
        jnp.dot(esm_ref[...], ws_ref[...],
                preferred_element_type=jnp.float32) + bs_ref[...])
    h0 = jnp.dot(r0, w1_ref[0], preferred_element_type=jnp.float32)
    h1 = jnp.dot(r1, w1_ref[1], preferred_element_type=jnp.float32)
    out_ref[...] = jax.nn.relu(h0 + h1 + b1_ref[...])


def _encoder(esm_x, evo_x, params):
    blk = 1000
    w1 = params['evo1']['w'].reshape(2, 2 * H, H)
    grid = (N // blk,)
    return pl.pallas_call(
        _encoder_body,
        grid=grid,
        in_specs=[
            pl.BlockSpec((blk, ESM), lambda i: (i, 0)),
            pl.BlockSpec((blk, EVO), lambda i: (i, 0)),
            pl.BlockSpec((ESM, 2 * H), lambda i: (0, 0)),
            pl.BlockSpec((2 * H,), lambda i: (0,)),
            pl.BlockSpec((EVO, 2 * H), lambda i: (0, 0)),
            pl.BlockSpec((2 * H,), lambda i: (0,)),
            pl.BlockSpec((2, 2 * H, H), lambda i: (0, 0, 0)),
            pl.BlockSpec((H,), lambda i: (0,)),
        ],
        out_specs=pl.BlockSpec((blk, H), lambda i: (i, 0)),
        out_shape=jax.ShapeDtypeStruct((N, H), jnp.float32),
    )(esm_x, evo_x, params['esm']['w'], params['esm']['b'],
      params['evo']['w'], params['evo']['b'], w1, params['evo1']['b'])


EB = 1000
NBLK = EPG // EB


def _agg_body(dl_ref, sl_ref, ea_ref, xt_ref, we_ref, be_ref,
              w1_ref, b1_ref, w2_ref, b2_ref,
              mn_ref, mx_ref, sm_ref, sq_ref, ct_ref):
    j = pl.program_id(1)

    @pl.when(j == 0)
    def _():
        mn_ref[...] = jnp.full((1, NPG, H), 3.0e38, jnp.float32)
        mx_ref[...] = jnp.full((1, NPG, H), -3.0e38, jnp.float32)
        sm_ref[...] = jnp.zeros((1, NPG, H), jnp.float32)
        sq_ref[...] = jnp.zeros((1, NPG, H), jnp.float32)
        ct_ref[...] = jnp.zeros((1, NPG, 1), jnp.float32)

    d = dl_ref[...]
    cols = jax.lax.broadcasted_iota(jnp.int32, (EB, NPG), 1)
    m = (d == cols).astype(jnp.float32)
    ms = (sl_ref[...] == cols).astype(jnp.float32)
    xt = xt_ref[0]
    xd = jax.lax.dot_general(m, xt, (((1,), (0,)), ((), ())),
                             preferred_element_type=jnp.float32)
    xs = jax.lax.dot_general(ms, xt, (((1,), (0,)), ((), ())),
                             preferred_element_type=jnp.float32)
    e = jax.lax.dot_general(ea_ref[...], we_ref[...], (((1,), (0,)), ((), ())),
                            preferred_element_type=jnp.float32) + be_ref[...]
    parts = []
    for t in range(T):
        h = jnp.concatenate([xd[:, t * F:(t + 1) * F],
                             xs[:, t * F:(t + 1) * F], e], axis=1)
        z = jax.lax.dot_general(h, w1_ref[t], (((1,), (0,)), ((), ())),
                                preferred_element_type=jnp.float32) + b1_ref[t]
        z = jax.lax.dot_general(jax.nn.relu(z), w2_ref[t],
                                (((1,), (0,)), ((), ())),
                                preferred_element_type=jnp.float32) + b2_ref[t]
        parts.append(z)
    msg = jnp.concatenate(parts, axis=1)
    x = msg
    y = msg
    rows = jax.lax.broadcasted_iota(jnp.int32, (EB, 1), 0)
    k = 1
    while k < EB:
        dk = pltpu.roll(d, k, axis=0)
        valid = (d == dk) & (rows >= k)
        x = jnp.where(valid, jnp.minimum(x, pltpu.roll(x, k, axis=0)), x)
        y = jnp.where(valid, jnp.maximum(y, pltpu.roll(y, k, axis=0)), y)
        k *= 2
    is_end = (d != pltpu.roll(d, EB - 1, axis=0)) | (rows == EB - 1)
    ef = is_end.astype(jnp.float32)
    ones = jnp.ones((EB, 1), jnp.float32)
    rhs = jnp.concatenate([msg, msg * msg, x * ef, y * ef, ef, ones], axis=1)
    contrib = jax.lax.dot_general(m, rhs, (((0,), (0,)), ((), ())),
                                  preferred_element_type=jnp.float32)
    pres = contrib[:, 4 * H:4 * H + 1] > 0.5
    sm_ref[0] = sm_ref[0] + contrib[:, :H]
    sq_ref[0] = sq_ref[0] + contrib[:, H:2 * H]
    ct_ref[0] = ct_ref[0] + contrib[:, 4 * H + 1:4 * H + 2]
    mn_ref[0] = jnp.where(pres, jnp.minimum(mn_ref[0], contrib[:, 2 * H:3 * H]),
                          mn_ref[0])
    mx_ref[0] = jnp.where(pres, jnp.maximum(mx_ref[0], contrib[:, 3 * H:4 * H]),
                          mx_ref[0])

    @pl.when(j == NBLK - 1)
    def _():
        mn_ref[0] = jnp.where(mn_ref[0] > 1.0e37, 0.0, mn_ref[0])
        mx_ref[0] = jnp.where(mx_ref[0] < -1.0e37, 0.0, mx_ref[0])


def _seg_agg(dstl_s, srcl_s, e_attr, xt3, lp):
    w1 = jnp.stack([lp['pre'][t][0]['w'] for t in range(T)])
    b1 = jnp.stack([lp['pre'][t][0]['b'] for t in range(T)])
    w2 = jnp.stack([lp['pre'][t][1]['w'] for t in range(T)])
    b2 = jnp.stack([lp['pre'][t][1]['b'] for t in range(T)])
    outs = pl.pallas_call(
        _agg_body,
        grid=(B, NBLK),
        in_specs=[
            pl.BlockSpec((EB, 1), lambda b, j: (b * NBLK + j, 0)),
            pl.BlockSpec((EB, 1), lambda b, j: (b * NBLK + j, 0)),
            pl.BlockSpec((EB, H), lambda b, j: (b * NBLK + j, 0)),
            pl.BlockSpec((1, NPG, H), lambda b, j: (b, 0, 0)),
            pl.BlockSpec((H, F), lambda b, j: (0, 0)),
            pl.BlockSpec((F,), lambda b, j: (0,)),
            pl.BlockSpec((T, 3 * F, F), lambda b, j: (0, 0, 0)),
            pl.BlockSpec((T, F), lambda b, j: (0, 0)),
            pl.BlockSpec((T, F, F), lambda b, j: (0, 0, 0)),
            pl.BlockSpec((T, F), lambda b, j: (0, 0)),
        ],
        out_specs=[
            pl.BlockSpec((1, NPG, H), lambda b, j: (b, 0, 0)),
            pl.BlockSpec((1, NPG, H), lambda b, j: (b, 0, 0)),
            pl.BlockSpec((1, NPG, H), lambda b, j: (b, 0, 0)),
            pl.BlockSpec((1, NPG, H), lambda b, j: (b, 0, 0)),
            pl.BlockSpec((1, NPG, 1), lambda b, j: (b, 0, 0)),
        ],
        out_shape=[jax.ShapeDtypeStruct((B, NPG, H), jnp.float32),
                   jax.ShapeDtypeStruct((B, NPG, H), jnp.float32),
                   jax.ShapeDtypeStruct((B, NPG, H), jnp.float32),
                   jax.ShapeDtypeStruct((B, NPG, H), jnp.float32),
                   jax.ShapeDtypeStruct((B, NPG, 1), jnp.float32)],
    )(dstl_s, srcl_s, e_attr, xt3,
      lp['edge']['w'], lp['edge']['b'], w1, b1, w2, b2)
    return outs


def _post_body(mn_ref, mx_ref, sm_ref, sq_ref, ct_ref, xt_ref,
               wa_ref, wbcd_ref, pb_ref, lw_ref, lb_ref,
               out_ref, enz_ref):
    ct = ct_ref[0]
    c = jnp.maximum(ct, 1.0)
    mean = sm_ref[0] / c
    msq = sq_ref[0] / c
    std = jnp.sqrt(jnp.maximum(msq - mean * mean, 0.0) + 1e-5)
    mn = mn_ref[0]
    mx = mx_ref[0]
    amp = jnp.log(c + 1.0) / AVG_LOG
    lin_s = c / AVG_LIN
    xt = xt_ref[0]
    ys = []
    for t in range(T):
        sl = slice(t * F, (t + 1) * F)
        a = jnp.concatenate([mean[:, sl], mn[:, sl], mx[:, sl], std[:, sl]],
                            axis=1)
        z = jax.lax.dot_general(a, wbcd_ref[t], (((1,), (0,)), ((), ())),
                                preferred_element_type=jnp.float32)
        y = jax.lax.dot_general(xt[:, sl], wa_ref[t], (((1,), (0,)), ((), ())),
                                preferred_element_type=jnp.float32)
        y = y + z[:, :F] + amp * z[:, F:2 * F] + lin_s * z[:, 2 * F:] + pb_ref[t]
        ys.append(y)
    yy = jnp.concatenate(ys, axis=1)
    xnew = jax.lax.dot_general(yy, lw_ref[...], (((1,), (0,)), ((), ())),
                               preferred_element_type=jnp.float32) + lb_ref[...]
    out_ref[0] = xnew
    enz_ref[0] = jnp.max(xnew, axis=0, keepdims=True)


def _pna_post(mn, mx, sm, sq, ct, xt3, lp):
    wa = jnp.stack([lp['post'][t]['w'][:F] for t in range(T)])
    wbcd = jnp.stack([jnp.concatenate(
        [lp['post'][t]['w'][F:5 * F],
         lp['post'][t]['w'][5 * F:9 * F],
         lp['post'][t]['w'][9 * F:13 * F]], axis=1) for t in range(T)])
    pb = jnp.stack([lp['post'][t]['b'] for t in range(T)])
    return pl.pallas_call(
        _post_body,
        grid=(B,),
        in_specs=[
            pl.BlockSpec((1, NPG, H), lambda b: (b, 0, 0)),
            pl.BlockSpec((1, NPG, H), lambda b: (b, 0, 0)),
            pl.BlockSpec((1, NPG, H), lambda b: (b, 0, 0)),
            pl.BlockSpec((1, NPG, H), lambda b: (b, 0, 0)),
            pl.BlockSpec((1, NPG, 1), lambda b: (b, 0, 0)),
            pl.BlockSpec((1, NPG, H), lambda b: (b, 0, 0)),
            pl.BlockSpec((T, F, F), lambda b: (0, 0, 0)),
            pl.BlockSpec((T, 4 * F, 3 * F), lambda b: (0, 0, 0)),
            pl.BlockSpec((T, F), lambda b: (0, 0)),
            pl.BlockSpec((H, H), lambda b: (0, 0)),
            pl.BlockSpec((H,), lambda b: (0,)),
        ],
        out_specs=[
            pl.BlockSpec((1, NPG, H), lambda b: (b, 0, 0)),
            pl.BlockSpec((1, 1, H), lambda b: (b, 0, 0)),
        ],
        out_shape=[jax.ShapeDtypeStruct((B, NPG, H), jnp.float32),
                   jax.ShapeDtypeStruct((B, 1, H), jnp.float32)],
    )(mn, mx, sm, sq, ct, xt3, wa, wbcd, pb, lp['lin']['w'], lp['lin']['b'])


def _adj_body(dl_ref, sl_ref, adj_ref):
    j = pl.program_id(1)

    @pl.when(j == 0)
    def _():
        adj_ref[...] = jnp.zeros((1, NPG, NPG), jnp.float32)

    cols = jax.lax.broadcasted_iota(jnp.int32, (EB, NPG), 1)
    ms = (sl_ref[...] == cols).astype(jnp.float32)
    md = (dl_ref[...] == cols).astype(jnp.float32)
    adj_ref[0] = adj_ref[0] + jax.lax.dot_general(
        ms, md, (((0,), (0,)), ((), ())), preferred_element_type=jnp.float32)


def _build_adj(dstl_s, srcl_s):
    return pl.pallas_call(
        _adj_body,
        grid=(B, NBLK),
        in_specs=[
            pl.BlockSpec((EB, 1), lambda b, j: (b * NBLK + j, 0)),
            pl.BlockSpec((EB, 1), lambda b, j: (b * NBLK + j, 0)),
        ],
        out_specs=pl.BlockSpec((1, NPG, NPG), lambda b, j: (b, 0, 0)),
        out_shape=jax.ShapeDtypeStruct((B, NPG, NPG), jnp.float32),
    )(dstl_s, srcl_s)


def _gcn_pair_body(x_ref, adj_ref, dinv_ref, w1_ref, b1_ref, w2_ref, b2_ref,
                   s_ref):
    xg = x_ref[0]
    adj = adj_ref[0]
    dinv = dinv_ref[0]
    d2 = dinv * dinv
    xw1 = jax.lax.dot_general(xg, w1_ref[...], (((1,), (0,)), ((), ())),
                              preferred_element_type=jnp.float32)
    agg1 = dinv * jax.lax.dot_general(adj, dinv * xw1,
                                      (((0,), (0,)), ((), ())),
                                      preferred_element_type=jnp.float32)
    hsig = jax.nn.relu(agg1 + xw1 * d2 + b1_ref[...])
    xw2 = jax.lax.dot_general(hsig, w2_ref[...], (((1,), (0,)), ((), ())),
                              preferred_element_type=jnp.float32)
    agg2 = dinv * jax.lax.dot_general(adj, dinv * xw2,
                                      (((0,), (0,)), ((), ())),
                                      preferred_element_type=jnp.float32)
    s_ref[0] = agg2 + xw2 * d2 + b2_ref[...]


def _gcn_pair(x3, adj, dinv3, lp, K):
    return pl.pallas_call(
        _gcn_pair_body,
        grid=(B,),
        in_specs=[
            pl.BlockSpec((1, NPG, H), lambda b: (b, 0, 0)),
            pl.BlockSpec((1, NPG, NPG), lambda b: (b, 0, 0)),
            pl.BlockSpec((1, NPG, 1), lambda b: (b, 0, 0)),
            pl.BlockSpec((H, 2 * H), lambda b: (0, 0)),
            pl.BlockSpec((2 * H,), lambda b: (0,)),
            pl.BlockSpec((2 * H, K), lambda b: (0, 0)),
            pl.BlockSpec((K,), lambda b: (0,)),
        ],
        out_specs=pl.BlockSpec((1, NPG, K), lambda b: (b, 0, 0)),
        out_shape=jax.ShapeDtypeStruct((B, NPG, K), jnp.float32),
    )(x3, adj, dinv3, lp['gcn1']['w'], lp['gcn1']['b'],
      lp['gcn2']['w'], lp['gcn2']['b'])


def _apply(p, x):
    return x @ p['w'] + p['b']


def _rbf(d):
    d = jnp.minimum(d, 1.0)
    mu = jnp.linspace(0.0, 1.0, H)
    sig = 1.0 / H
    return jnp.exp(-(((d[:, None] - mu[None, :]) / sig) ** 2))


def _layer_norm(x, g, b):
    m = x.mean(-1, keepdims=True)
    v = ((x - m) ** 2).mean(-1, keepdims=True)
    return g * (x - m) / jnp.sqrt(v + 1e-5) + b


def _graph_norm(x, params):
    xr = x.reshape(B, NPG, H)
    mean = xr.mean(1, keepdims=True)
    out = xr - params['gn_ms'] * mean
    var = (out ** 2).mean(1, keepdims=True)
    out = out / jnp.sqrt(var + 1e-5)
    return (params['gn_w'] * out + params['gn_b']).reshape(N, H)


def _pna(p, x, dstl_s, srcl_s, e_attr):
    xt3 = x.reshape(B, NPG, H)
    mn, mx, sm, sq, ct = _seg_agg(dstl_s, srcl_s, e_attr, xt3, p)
    xnew, enz1 = _pna_post(mn, mx, sm, sq, ct, xt3, p)
    return xnew.reshape(N, H), enz1.reshape(B, H)


def _gcn(x, p, src, dst, dinv):
    xw = x @ p['w']
    nrm = (dinv[src] * dinv[dst])[:, None]
    out = jax.ops.segment_sum(xw[src] * nrm, dst, num_segments=N) \
        + xw * (dinv * dinv)[:, None]
    return out + p['b']


def _mincut(x, adj, s, K):
    s = jax.nn.softmax(s, -1)
    out = jnp.einsum('bnk,bnd->bkd', s, x)
    sa = jnp.einsum('bnk,bnm->bkm', s, adj)
    out_adj = jnp.einsum('bkm,bml->bkl', sa, s)
    num = jnp.trace(out_adj, axis1=1, axis2=2)
    d_flat = adj.sum(-1)
    den = (d_flat[..., None] * s * s).sum((1, 2))
    mincut_loss = -jnp.mean(num / (den + 1e-15))
    ss = jnp.einsum('bnk,bnl->bkl', s, s)
    i_s = jnp.eye(K, dtype=jnp.float32)
    ortho = jnp.linalg.norm(
        ss / jnp.linalg.norm(ss, axis=(-1, -2), keepdims=True)
        - i_s / jnp.linalg.norm(i_s), axis=(-1, -2))
    ortho_loss = jnp.mean(ortho)
    out_adj = out_adj * (1.0 - i_s)
    dd = jnp.sqrt(out_adj.sum(-1))[..., None] + 1e-15
    out_adj = out_adj / dd / jnp.transpose(dd, (0, 2, 1))
    return s, out, out_adj, mincut_loss, ortho_loss


def kernel(residue_x, residue_evo_x, residue_edge_index, residue_edge_weight,
           prot_batch, params):
    dst, src, ew = jax.lax.sort(
        (residue_edge_index[1], residue_edge_index[0], residue_edge_weight),
        num_keys=1)
    dstl_s = (dst % NPG).astype(jnp.int32)[:, None]
    srcl_s = (src % NPG).astype(jnp.int32)[:, None]
    e_attr = _rbf(ew)
    x = _encoder(residue_x, residue_evo_x, params)
    adj = _build_adj(dstl_s, srcl_s)
    deg = adj.sum(1).reshape(N)
    dinv = 1.0 / jnp.sqrt(deg + 1.0)
    dinv3 = dinv.reshape(B, NPG, 1)
    ln = lambda z: _layer_norm(z, params['ln_g'], params['ln_b'])
    enz1 = []; enz2 = []; clus = []
    ortho_loss = jnp.asarray(0.0, jnp.float32)
    cluster_loss = jnp.asarray(0.0, jnp.float32)
    for idx, lp in enumerate(params['layers']):
        K = KS[idx]
        x = _graph_norm(x, params)
        x, e1v = _pna(lp, x, dstl_s, srcl_s, e_attr)
        enz1.append(e1v)
        s_d = _gcn_pair(x.reshape(B, NPG, H), adj, dinv3, lp, K)
        hx = x.reshape(B, NPG, H)
        s_sm, cx, _, cl, ol = _mincut(hx, adj, s_d, K)
        cx = ln(cx)
        ortho_loss = ortho_loss + ol
        cluster_loss = cluster_loss + cl
        clus.append(ln(cx.max(1)))
        x = x + jax.nn.relu(_apply(lp['res'],
                                   jnp.einsum('bnk,bkd->bnd', s_sm, cx).reshape(N, H)))
        enz2.append(ln(x.reshape(B, NPG, H).max(1)))
    e1 = jax.nn.relu(_apply(params['mol1'], jnp.concatenate(enz1, -1)))
    e2 = jax.nn.relu(_apply(params['mol2'], jnp.concatenate(enz2, -1)))
    cf = jax.nn.relu(_apply(params['mol3'], jnp.concatenate(clus, -1)))
    feat = jnp.concatenate([e1, e2, cf], -1)
    h0 = jax.nn.relu(_apply(params['cls0'], feat))
    h1 = jax.nn.relu(_apply(params['cls1'], h0))
    reg = _apply(params['cls2'], h1)
    return (reg, jnp.asarray(0.0, jnp.float32), ortho_loss, cluster_loss)


# speedup vs baseline: 1.0529x; 1.0529x over previous
import functools

import jax
import jax.numpy as jnp
import numpy as np
from jax.experimental import pallas as pl
from jax.experimental.pallas import tpu as pltpu

B = 20; NPG = 500; N = B * NPG
EPG = 8000; E = B * EPG
H = 200; T = 5; F = 40
KS = [3, 10, 30]
ESM = 1280; EVO = 1024
NUM_CLASS = 384
DEG_HIST = np.zeros(33); DEG_HIST[16] = N
_bins = np.arange(33).astype(np.float64)
AVG_LIN = float((_bins * DEG_HIST).sum() / DEG_HIST.sum())
AVG_LOG = float((np.log(_bins + 1.0) * DEG_HIST).sum() / DEG_HIST.sum())


def _encoder_body(esm_ref, evo_ref, ws_ref, bs_ref, we_ref, be_ref,
                  w1_ref, b1_ref, out_ref):
    r0 = jax.nn.relu(
        jnp.dot(evo_ref[...], we_ref[...],
                preferred_element_type=jnp.float32) + be_ref[...])
    r1 = jax.nn.relu(
        jnp.dot(esm_ref[...], ws_ref[...],
                preferred_element_type=jnp.float32) + bs_ref[...])
    h0 = jnp.dot(r0, w1_ref[0], preferred_element_type=jnp.float32)
    h1 = jnp.dot(r1, w1_ref[1], preferred_element_type=jnp.float32)
    out_ref[...] = jax.nn.relu(h0 + h1 + b1_ref[...])


def _encoder(esm_x, evo_x, params):
    blk = 1000
    w1 = params['evo1']['w'].reshape(2, 2 * H, H)
    grid = (N // blk,)
    return pl.pallas_call(
        _encoder_body,
        grid=grid,
        in_specs=[
            pl.BlockSpec((blk, ESM), lambda i: (i, 0)),
            pl.BlockSpec((blk, EVO), lambda i: (i, 0)),
            pl.BlockSpec((ESM, 2 * H), lambda i: (0, 0)),
            pl.BlockSpec((2 * H,), lambda i: (0,)),
            pl.BlockSpec((EVO, 2 * H), lambda i: (0, 0)),
            pl.BlockSpec((2 * H,), lambda i: (0,)),
            pl.BlockSpec((2, 2 * H, H), lambda i: (0, 0, 0)),
            pl.BlockSpec((H,), lambda i: (0,)),
        ],
        out_specs=pl.BlockSpec((blk, H), lambda i: (i, 0)),
        out_shape=jax.ShapeDtypeStruct((N, H), jnp.float32),
    )(esm_x, evo_x, params['esm']['w'], params['esm']['b'],
      params['evo']['w'], params['evo']['b'], w1, params['evo1']['b'])


EB = 1000
NBLK = EPG // EB


def _agg_body(dl_ref, sl_ref, ea_ref, xt_ref, we_ref, be_ref,
              w1_ref, b1_ref, w2_ref, b2_ref,
              mn_ref, mx_ref, sm_ref, sq_ref, ct_ref, x_ref, y_ref):
    j = pl.program_id(1)

    @pl.when(j == 0)
    def _():
        mn_ref[...] = jnp.full((1, NPG, H), 3.0e38, jnp.float32)
        mx_ref[...] = jnp.full((1, NPG, H), -3.0e38, jnp.float32)
        sm_ref[...] = jnp.zeros((1, NPG, H), jnp.float32)
        sq_ref[...] = jnp.zeros((1, NPG, H), jnp.float32)
        ct_ref[...] = jnp.zeros((1, NPG, 1), jnp.float32)

    d = dl_ref[...]
    cols = jax.lax.broadcasted_iota(jnp.int32, (EB, NPG), 1)
    m = (d == cols).astype(jnp.float32)
    ms = (sl_ref[...] == cols).astype(jnp.float32)
    xt = xt_ref[0]
    xd = jax.lax.dot_general(m, xt, (((1,), (0,)), ((), ())),
                             preferred_element_type=jnp.float32)
    xs = jax.lax.dot_general(ms, xt, (((1,), (0,)), ((), ())),
                             preferred_element_type=jnp.float32)
    e = jax.lax.dot_general(ea_ref[...], we_ref[...], (((1,), (0,)), ((), ())),
                            preferred_element_type=jnp.float32) + be_ref[...]
    parts = []
    for t in range(T):
        h = jnp.concatenate([xd[:, t * F:(t + 1) * F],
                             xs[:, t * F:(t + 1) * F], e], axis=1)
        z = jax.lax.dot_general(h, w1_ref[t], (((1,), (0,)), ((), ())),
                                preferred_element_type=jnp.float32) + b1_ref[t]
        z = jax.lax.dot_general(jax.nn.relu(z), w2_ref[t],
                                (((1,), (0,)), ((), ())),
                                preferred_element_type=jnp.float32) + b2_ref[t]
        parts.append(z)
    msg = jnp.concatenate(parts, axis=1)
    x_ref[...] = msg
    y_ref[...] = msg
    rows = jax.lax.broadcasted_iota(jnp.int32, (EB, 1), 0)
    k = 1
    while k < EB:
        dk = pltpu.roll(d, k, axis=0)
        valid = (d == dk) & (rows >= k)

        def _step(valid=valid, k=k):
            xs = pltpu.roll(x_ref[...], k, axis=0)
            ys = pltpu.roll(y_ref[...], k, axis=0)
            x_ref[...] = jnp.where(valid, jnp.minimum(x_ref[...], xs),
                                   x_ref[...])
            y_ref[...] = jnp.where(valid, jnp.maximum(y_ref[...], ys),
                                   y_ref[...])

        if k >= 64:
            pl.when(jnp.any(valid))(_step)
        else:
            _step()
        k *= 2
    x = x_ref[...]
    y = y_ref[...]
    is_end = (d != pltpu.roll(d, EB - 1, axis=0)) | (rows == EB - 1)
    ef = is_end.astype(jnp.float32)
    ones = jnp.ones((EB, 1), jnp.float32)
    rhs = jnp.concatenate([msg, msg * msg, x * ef, y * ef, ef, ones], axis=1)
    contrib = jax.lax.dot_general(m, rhs, (((0,), (0,)), ((), ())),
                                  preferred_element_type=jnp.float32)
    pres = contrib[:, 4 * H:4 * H + 1] > 0.5
    sm_ref[0] = sm_ref[0] + contrib[:, :H]
    sq_ref[0] = sq_ref[0] + contrib[:, H:2 * H]
    ct_ref[0] = ct_ref[0] + contrib[:, 4 * H + 1:4 * H + 2]
    mn_ref[0] = jnp.where(pres, jnp.minimum(mn_ref[0], contrib[:, 2 * H:3 * H]),
                          mn_ref[0])
    mx_ref[0] = jnp.where(pres, jnp.maximum(mx_ref[0], contrib[:, 3 * H:4 * H]),
                          mx_ref[0])

    @pl.when(j == NBLK - 1)
    def _():
        mn_ref[0] = jnp.where(mn_ref[0] > 1.0e37, 0.0, mn_ref[0])
        mx_ref[0] = jnp.where(mx_ref[0] < -1.0e37, 0.0, mx_ref[0])


def _seg_agg(dstl_s, srcl_s, e_attr, xt3, lp):
    w1 = jnp.stack([lp['pre'][t][0]['w'] for t in range(T)])
    b1 = jnp.stack([lp['pre'][t][0]['b'] for t in range(T)])
    w2 = jnp.stack([lp['pre'][t][1]['w'] for t in range(T)])
    b2 = jnp.stack([lp['pre'][t][1]['b'] for t in range(T)])
    outs = pl.pallas_call(
        _agg_body,
        grid=(B, NBLK),
        in_specs=[
            pl.BlockSpec((EB, 1), lambda b, j: (b * NBLK + j, 0)),
            pl.BlockSpec((EB, 1), lambda b, j: (b * NBLK + j, 0)),
            pl.BlockSpec((EB, H), lambda b, j: (b * NBLK + j, 0)),
            pl.BlockSpec((1, NPG, H), lambda b, j: (b, 0, 0)),
            pl.BlockSpec((H, F), lambda b, j: (0, 0)),
            pl.BlockSpec((F,), lambda b, j: (0,)),
            pl.BlockSpec((T, 3 * F, F), lambda b, j: (0, 0, 0)),
            pl.BlockSpec((T, F), lambda b, j: (0, 0)),
            pl.BlockSpec((T, F, F), lambda b, j: (0, 0, 0)),
            pl.BlockSpec((T, F), lambda b, j: (0, 0)),
        ],
        out_specs=[
            pl.BlockSpec((1, NPG, H), lambda b, j: (b, 0, 0)),
            pl.BlockSpec((1, NPG, H), lambda b, j: (b, 0, 0)),
            pl.BlockSpec((1, NPG, H), lambda b, j: (b, 0, 0)),
            pl.BlockSpec((1, NPG, H), lambda b, j: (b, 0, 0)),
            pl.BlockSpec((1, NPG, 1), lambda b, j: (b, 0, 0)),
        ],
        out_shape=[jax.ShapeDtypeStruct((B, NPG, H), jnp.float32),
                   jax.ShapeDtypeStruct((B, NPG, H), jnp.float32),
                   jax.ShapeDtypeStruct((B, NPG, H), jnp.float32),
                   jax.ShapeDtypeStruct((B, NPG, H), jnp.float32),
                   jax.ShapeDtypeStruct((B, NPG, 1), jnp.float32)],
        scratch_shapes=[pltpu.VMEM((EB, H), jnp.float32),
                        pltpu.VMEM((EB, H), jnp.float32)],
    )(dstl_s, srcl_s, e_attr, xt3,
      lp['edge']['w'], lp['edge']['b'], w1, b1, w2, b2)
    return outs


def _post_body(mn_ref, mx_ref, sm_ref, sq_ref, ct_ref, xt_ref,
               wa_ref, wbcd_ref, pb_ref, lw_ref, lb_ref,
               out_ref, enz_ref):
    ct = ct_ref[0]
    c = jnp.maximum(ct, 1.0)
    mean = sm_ref[0] / c
    msq = sq_ref[0] / c
    std = jnp.sqrt(jnp.maximum(msq - mean * mean, 0.0) + 1e-5)
    mn = mn_ref[0]
    mx = mx_ref[0]
    amp = jnp.log(c + 1.0) / AVG_LOG
    lin_s = c / AVG_LIN
    xt = xt_ref[0]
    ys = []
    for t in range(T):
        sl = slice(t * F, (t + 1) * F)
        a = jnp.concatenate([mean[:, sl], mn[:, sl], mx[:, sl], std[:, sl]],
                            axis=1)
        z = jax.lax.dot_general(a, wbcd_ref[t], (((1,), (0,)), ((), ())),
                                preferred_element_type=jnp.float32)
        y = jax.lax.dot_general(xt[:, sl], wa_ref[t], (((1,), (0,)), ((), ())),
                                preferred_element_type=jnp.float32)
        y = y + z[:, :F] + amp * z[:, F:2 * F] + lin_s * z[:, 2 * F:] + pb_ref[t]
        ys.append(y)
    yy = jnp.concatenate(ys, axis=1)
    xnew = jax.lax.dot_general(yy, lw_ref[...], (((1,), (0,)), ((), ())),
                               preferred_element_type=jnp.float32) + lb_ref[...]
    out_ref[0] = xnew
    enz_ref[0] = jnp.max(xnew, axis=0, keepdims=True)


def _pna_post(mn, mx, sm, sq, ct, xt3, lp):
    wa = jnp.stack([lp['post'][t]['w'][:F] for t in range(T)])
    wbcd = jnp.stack([jnp.concatenate(
        [lp['post'][t]['w'][F:5 * F],
         lp['post'][t]['w'][5 * F:9 * F],
         lp['post'][t]['w'][9 * F:13 * F]], axis=1) for t in range(T)])
    pb = jnp.stack([lp['post'][t]['b'] for t in range(T)])
    return pl.pallas_call(
        _post_body,
        grid=(B,),
        in_specs=[
            pl.BlockSpec((1, NPG, H), lambda b: (b, 0, 0)),
            pl.BlockSpec((1, NPG, H), lambda b: (b, 0, 0)),
            pl.BlockSpec((1, NPG, H), lambda b: (b, 0, 0)),
            pl.BlockSpec((1, NPG, H), lambda b: (b, 0, 0)),
            pl.BlockSpec((1, NPG, 1), lambda b: (b, 0, 0)),
            pl.BlockSpec((1, NPG, H), lambda b: (b, 0, 0)),
            pl.BlockSpec((T, F, F), lambda b: (0, 0, 0)),
            pl.BlockSpec((T, 4 * F, 3 * F), lambda b: (0, 0, 0)),
            pl.BlockSpec((T, F), lambda b: (0, 0)),
            pl.BlockSpec((H, H), lambda b: (0, 0)),
            pl.BlockSpec((H,), lambda b: (0,)),
        ],
        out_specs=[
            pl.BlockSpec((1, NPG, H), lambda b: (b, 0, 0)),
            pl.BlockSpec((1, 1, H), lambda b: (b, 0, 0)),
        ],
        out_shape=[jax.ShapeDtypeStruct((B, NPG, H), jnp.float32),
                   jax.ShapeDtypeStruct((B, 1, H), jnp.float32)],
    )(mn, mx, sm, sq, ct, xt3, wa, wbcd, pb, lp['lin']['w'], lp['lin']['b'])


def _adj_body(dl_ref, sl_ref, adj_ref):
    j = pl.program_id(1)

    @pl.when(j == 0)
    def _():
        adj_ref[...] = jnp.zeros((1, NPG, NPG), jnp.float32)

    cols = jax.lax.broadcasted_iota(jnp.int32, (EB, NPG), 1)
    ms = (sl_ref[...] == cols).astype(jnp.float32)
    md = (dl_ref[...] == cols).astype(jnp.float32)
    adj_ref[0] = adj_ref[0] + jax.lax.dot_general(
        ms, md, (((0,), (0,)), ((), ())), preferred_element_type=jnp.float32)


def _build_adj(dstl_s, srcl_s):
    return pl.pallas_call(
        _adj_body,
        grid=(B, NBLK),
        in_specs=[
            pl.BlockSpec((EB, 1), lambda b, j: (b * NBLK + j, 0)),
            pl.BlockSpec((EB, 1), lambda b, j: (b * NBLK + j, 0)),
        ],
        out_specs=pl.BlockSpec((1, NPG, NPG), lambda b, j: (b, 0, 0)),
        out_shape=jax.ShapeDtypeStruct((B, NPG, NPG), jnp.float32),
    )(dstl_s, srcl_s)


def _gcn_pair_body(x_ref, adj_ref, dinv_ref, w1_ref, b1_ref, w2_ref, b2_ref,
                   s_ref):
    xg = x_ref[0]
    adj = adj_ref[0]
    dinv = dinv_ref[0]
    d2 = dinv * dinv
    xw1 = jax.lax.dot_general(xg, w1_ref[...], (((1,), (0,)), ((), ())),
                              preferred_element_type=jnp.float32)
    agg1 = dinv * jax.lax.dot_general(adj, dinv * xw1,
                                      (((0,), (0,)), ((), ())),
                                      preferred_element_type=jnp.float32)
    hsig = jax.nn.relu(agg1 + xw1 * d2 + b1_ref[...])
    xw2 = jax.lax.dot_general(hsig, w2_ref[...], (((1,), (0,)), ((), ())),
                              preferred_element_type=jnp.float32)
    agg2 = dinv * jax.lax.dot_general(adj, dinv * xw2,
                                      (((0,), (0,)), ((), ())),
                                      preferred_element_type=jnp.float32)
    s_ref[0] = agg2 + xw2 * d2 + b2_ref[...]


def _gcn_pair(x3, adj, dinv3, lp, K):
    return pl.pallas_call(
        _gcn_pair_body,
        grid=(B,),
        in_specs=[
            pl.BlockSpec((1, NPG, H), lambda b: (b, 0, 0)),
            pl.BlockSpec((1, NPG, NPG), lambda b: (b, 0, 0)),
            pl.BlockSpec((1, NPG, 1), lambda b: (b, 0, 0)),
            pl.BlockSpec((H, 2 * H), lambda b: (0, 0)),
            pl.BlockSpec((2 * H,), lambda b: (0,)),
            pl.BlockSpec((2 * H, K), lambda b: (0, 0)),
            pl.BlockSpec((K,), lambda b: (0,)),
        ],
        out_specs=pl.BlockSpec((1, NPG, K), lambda b: (b, 0, 0)),
        out_shape=jax.ShapeDtypeStruct((B, NPG, K), jnp.float32),
    )(x3, adj, dinv3, lp['gcn1']['w'], lp['gcn1']['b'],
      lp['gcn2']['w'], lp['gcn2']['b'])


def _apply(p, x):
    return x @ p['w'] + p['b']


def _rbf(d):
    d = jnp.minimum(d, 1.0)
    mu = jnp.linspace(0.0, 1.0, H)
    sig = 1.0 / H
    return jnp.exp(-(((d[:, None] - mu[None, :]) / sig) ** 2))


def _layer_norm(x, g, b):
    m = x.mean(-1, keepdims=True)
    v = ((x - m) ** 2).mean(-1, keepdims=True)
    return g * (x - m) / jnp.sqrt(v + 1e-5) + b


def _graph_norm(x, params):
    xr = x.reshape(B, NPG, H)
    mean = xr.mean(1, keepdims=True)
    out = xr - params['gn_ms'] * mean
    var = (out ** 2).mean(1, keepdims=True)
    out = out / jnp.sqrt(var + 1e-5)
    return (params['gn_w'] * out + params['gn_b']).reshape(N, H)


def _pna(p, x, dstl_s, srcl_s, e_attr):
    xt3 = x.reshape(B, NPG, H)
    mn, mx, sm, sq, ct = _seg_agg(dstl_s, srcl_s, e_attr, xt3, p)
    xnew, enz1 = _pna_post(mn, mx, sm, sq, ct, xt3, p)
    return xnew.reshape(N, H), enz1.reshape(B, H)


def _gcn(x, p, src, dst, dinv):
    xw = x @ p['w']
    nrm = (dinv[src] * dinv[dst])[:, None]
    out = jax.ops.segment_sum(xw[src] * nrm, dst, num_segments=N) \
        + xw * (dinv * dinv)[:, None]
    return out + p['b']


def _mincut(x, adj, s, K):
    s = jax.nn.softmax(s, -1)
    out = jnp.einsum('bnk,bnd->bkd', s, x)
    sa = jnp.einsum('bnk,bnm->bkm', s, adj)
    out_adj = jnp.einsum('bkm,bml->bkl', sa, s)
    num = jnp.trace(out_adj, axis1=1, axis2=2)
    d_flat = adj.sum(-1)
    den = (d_flat[..., None] * s * s).sum((1, 2))
    mincut_loss = -jnp.mean(num / (den + 1e-15))
    ss = jnp.einsum('bnk,bnl->bkl', s, s)
    i_s = jnp.eye(K, dtype=jnp.float32)
    ortho = jnp.linalg.norm(
        ss / jnp.linalg.norm(ss, axis=(-1, -2), keepdims=True)
        - i_s / jnp.linalg.norm(i_s), axis=(-1, -2))
    ortho_loss = jnp.mean(ortho)
    out_adj = out_adj * (1.0 - i_s)
    dd = jnp.sqrt(out_adj.sum(-1))[..., None] + 1e-15
    out_adj = out_adj / dd / jnp.transpose(dd, (0, 2, 1))
    return s, out, out_adj, mincut_loss, ortho_loss


def kernel(residue_x, residue_evo_x, residue_edge_index, residue_edge_weight,
           prot_batch, params):
    perm = jnp.argsort(residue_edge_index[1])
    src = residue_edge_index[0][perm]
    dst = residue_edge_index[1][perm]
    ew = residue_edge_weight[perm]
    dstl_s = (dst % NPG).astype(jnp.int32)[:, None]
    srcl_s = (src % NPG).astype(jnp.int32)[:, None]
    e_attr = _rbf(ew)
    x = _encoder(residue_x, residue_evo_x, params)
    adj = _build_adj(dstl_s, srcl_s)
    deg = adj.sum(1).reshape(N)
    dinv = 1.0 / jnp.sqrt(deg + 1.0)
    dinv3 = dinv.reshape(B, NPG, 1)
    ln = lambda z: _layer_norm(z, params['ln_g'], params['ln_b'])
    enz1 = []; enz2 = []; clus = []
    ortho_loss = jnp.asarray(0.0, jnp.float32)
    cluster_loss = jnp.asarray(0.0, jnp.float32)
    for idx, lp in enumerate(params['layers']):
        K = KS[idx]
        x = _graph_norm(x, params)
        x, e1v = _pna(lp, x, dstl_s, srcl_s, e_attr)
        enz1.append(e1v)
        s_d = _gcn_pair(x.reshape(B, NPG, H), adj, dinv3, lp, K)
        hx = x.reshape(B, NPG, H)
        s_sm, cx, _, cl, ol = _mincut(hx, adj, s_d, K)
        cx = ln(cx)
        ortho_loss = ortho_loss + ol
        cluster_loss = cluster_loss + cl
        clus.append(ln(cx.max(1)))
        x = x + jax.nn.relu(_apply(lp['res'],
                                   jnp.einsum('bnk,bkd->bnd', s_sm, cx).reshape(N, H)))
        enz2.append(ln(x.reshape(B, NPG, H).max(1)))
    e1 = jax.nn.relu(_apply(params['mol1'], jnp.concatenate(enz1, -1)))
    e2 = jax.nn.relu(_apply(params['mol2'], jnp.concatenate(enz2, -1)))
    cf = jax.nn.relu(_apply(params['mol3'], jnp.concatenate(clus, -1)))
    feat = jnp.concatenate([e1, e2, cf], -1)
    h0 = jax.nn.relu(_apply(params['cls0'], feat))
    h1 = jax.nn.relu(_apply(params['cls1'], h0))
    reg = _apply(params['cls2'], h1)
    return (reg, jnp.asarray(0.0, jnp.float32), ortho_loss, cluster_loss)


# speedup vs baseline: 1.0772x; 1.0231x over previous
import functools

import jax
import jax.numpy as jnp
import numpy as np
from jax.experimental import pallas as pl
from jax.experimental.pallas import tpu as pltpu

B = 20; NPG = 500; N = B * NPG
EPG = 8000; E = B * EPG
H = 200; T = 5; F = 40
KS = [3, 10, 30]
ESM = 1280; EVO = 1024
NUM_CLASS = 384
DEG_HIST = np.zeros(33); DEG_HIST[16] = N
_bins = np.arange(33).astype(np.float64)
AVG_LIN = float((_bins * DEG_HIST).sum() / DEG_HIST.sum())
AVG_LOG = float((np.log(_bins + 1.0) * DEG_HIST).sum() / DEG_HIST.sum())


def _encoder_body(esm_ref, evo_ref, ws_ref, bs_ref, we_ref, be_ref,
                  w1_ref, b1_ref, out_ref):
    r0 = jax.nn.relu(
        jnp.dot(evo_ref[...], we_ref[...],
                preferred_element_type=jnp.float32) + be_ref[...])
    r1 = jax.nn.relu(
        jnp.dot(esm_ref[...], ws_ref[...],
                preferred_element_type=jnp.float32) + bs_ref[...])
    h0 = jnp.dot(r0, w1_ref[0], preferred_element_type=jnp.float32)
    h1 = jnp.dot(r1, w1_ref[1], preferred_element_type=jnp.float32)
    out_ref[...] = jax.nn.relu(h0 + h1 + b1_ref[...])


def _encoder(esm_x, evo_x, params):
    blk = 1000
    w1 = params['evo1']['w'].reshape(2, 2 * H, H)
    grid = (N // blk,)
    return pl.pallas_call(
        _encoder_body,
        grid=grid,
        in_specs=[
            pl.BlockSpec((blk, ESM), lambda i: (i, 0)),
            pl.BlockSpec((blk, EVO), lambda i: (i, 0)),
            pl.BlockSpec((ESM, 2 * H), lambda i: (0, 0)),
            pl.BlockSpec((2 * H,), lambda i: (0,)),
            pl.BlockSpec((EVO, 2 * H), lambda i: (0, 0)),
            pl.BlockSpec((2 * H,), lambda i: (0,)),
            pl.BlockSpec((2, 2 * H, H), lambda i: (0, 0, 0)),
            pl.BlockSpec((H,), lambda i: (0,)),
        ],
        out_specs=pl.BlockSpec((blk, H), lambda i: (i, 0)),
        out_shape=jax.ShapeDtypeStruct((N, H), jnp.float32),
    )(esm_x, evo_x, params['esm']['w'], params['esm']['b'],
      params['evo']['w'], params['evo']['b'], w1, params['evo1']['b'])


EB = 2000
NBLK = EPG // EB


def _agg_body(dl_ref, sl_ref, ea_ref, xt_ref, we_ref, be_ref,
              w1_ref, b1_ref, w2_ref, b2_ref,
              mn_ref, mx_ref, sm_ref, sq_ref, ct_ref, x_ref, y_ref):
    j = pl.program_id(1)

    @pl.when(j == 0)
    def _():
        mn_ref[...] = jnp.full((1, NPG, H), 3.0e38, jnp.float32)
        mx_ref[...] = jnp.full((1, NPG, H), -3.0e38, jnp.float32)
        sm_ref[...] = jnp.zeros((1, NPG, H), jnp.float32)
        sq_ref[...] = jnp.zeros((1, NPG, H), jnp.float32)
        ct_ref[...] = jnp.zeros((1, NPG, 1), jnp.float32)

    d = dl_ref[...]
    cols = jax.lax.broadcasted_iota(jnp.int32, (EB, NPG), 1)
    m = (d == cols).astype(jnp.float32)
    ms = (sl_ref[...] == cols).astype(jnp.float32)
    xt = xt_ref[0]
    xd = jax.lax.dot_general(m, xt, (((1,), (0,)), ((), ())),
                             preferred_element_type=jnp.float32)
    xs = jax.lax.dot_general(ms, xt, (((1,), (0,)), ((), ())),
                             preferred_element_type=jnp.float32)
    e = jax.lax.dot_general(ea_ref[...], we_ref[...], (((1,), (0,)), ((), ())),
                            preferred_element_type=jnp.float32) + be_ref[...]
    parts = []
    for t in range(T):
        h = jnp.concatenate([xd[:, t * F:(t + 1) * F],
                             xs[:, t * F:(t + 1) * F], e], axis=1)
        z = jax.lax.dot_general(h, w1_ref[t], (((1,), (0,)), ((), ())),
                                preferred_element_type=jnp.float32) + b1_ref[t]
        z = jax.lax.dot_general(jax.nn.relu(z), w2_ref[t],
                                (((1,), (0,)), ((), ())),
                                preferred_element_type=jnp.float32) + b2_ref[t]
        parts.append(z)
    msg = jnp.concatenate(parts, axis=1)
    x_ref[...] = msg
    y_ref[...] = msg
    rows = jax.lax.broadcasted_iota(jnp.int32, (EB, 1), 0)
    k = 1
    while k < EB:
        dk = pltpu.roll(d, k, axis=0)
        valid = (d == dk) & (rows >= k)

        def _step(valid=valid, k=k):
            xs = pltpu.roll(x_ref[...], k, axis=0)
            ys = pltpu.roll(y_ref[...], k, axis=0)
            x_ref[...] = jnp.where(valid, jnp.minimum(x_ref[...], xs),
                                   x_ref[...])
            y_ref[...] = jnp.where(valid, jnp.maximum(y_ref[...], ys),
                                   y_ref[...])

        if k >= 64:
            pl.when(jnp.any(valid))(_step)
        else:
            _step()
        k *= 2
    x = x_ref[...]
    y = y_ref[...]
    is_end = (d != pltpu.roll(d, EB - 1, axis=0)) | (rows == EB - 1)
    ef = is_end.astype(jnp.float32)
    ones = jnp.ones((EB, 1), jnp.float32)
    rhs = jnp.concatenate([msg, msg * msg, x * ef, y * ef, ef, ones], axis=1)
    contrib = jax.lax.dot_general(m, rhs, (((0,), (0,)), ((), ())),
                                  preferred_element_type=jnp.float32)
    pres = contrib[:, 4 * H:4 * H + 1] > 0.5
    sm_ref[0] = sm_ref[0] + contrib[:, :H]
    sq_ref[0] = sq_ref[0] + contrib[:, H:2 * H]
    ct_ref[0] = ct_ref[0] + contrib[:, 4 * H + 1:4 * H + 2]
    mn_ref[0] = jnp.where(pres, jnp.minimum(mn_ref[0], contrib[:, 2 * H:3 * H]),
                          mn_ref[0])
    mx_ref[0] = jnp.where(pres, jnp.maximum(mx_ref[0], contrib[:, 3 * H:4 * H]),
                          mx_ref[0])

    @pl.when(j == NBLK - 1)
    def _():
        mn_ref[0] = jnp.where(mn_ref[0] > 1.0e37, 0.0, mn_ref[0])
        mx_ref[0] = jnp.where(mx_ref[0] < -1.0e37, 0.0, mx_ref[0])


def _seg_agg(dstl_s, srcl_s, e_attr, xt3, lp):
    w1 = jnp.stack([lp['pre'][t][0]['w'] for t in range(T)])
    b1 = jnp.stack([lp['pre'][t][0]['b'] for t in range(T)])
    w2 = jnp.stack([lp['pre'][t][1]['w'] for t in range(T)])
    b2 = jnp.stack([lp['pre'][t][1]['b'] for t in range(T)])
    outs = pl.pallas_call(
        _agg_body,
        grid=(B, NBLK),
        in_specs=[
            pl.BlockSpec((EB, 1), lambda b, j: (b * NBLK + j, 0)),
            pl.BlockSpec((EB, 1), lambda b, j: (b * NBLK + j, 0)),
            pl.BlockSpec((EB, H), lambda b, j: (b * NBLK + j, 0)),
            pl.BlockSpec((1, NPG, H), lambda b, j: (b, 0, 0)),
            pl.BlockSpec((H, F), lambda b, j: (0, 0)),
            pl.BlockSpec((F,), lambda b, j: (0,)),
            pl.BlockSpec((T, 3 * F, F), lambda b, j: (0, 0, 0)),
            pl.BlockSpec((T, F), lambda b, j: (0, 0)),
            pl.BlockSpec((T, F, F), lambda b, j: (0, 0, 0)),
            pl.BlockSpec((T, F), lambda b, j: (0, 0)),
        ],
        out_specs=[
            pl.BlockSpec((1, NPG, H), lambda b, j: (b, 0, 0)),
            pl.BlockSpec((1, NPG, H), lambda b, j: (b, 0, 0)),
            pl.BlockSpec((1, NPG, H), lambda b, j: (b, 0, 0)),
            pl.BlockSpec((1, NPG, H), lambda b, j: (b, 0, 0)),
            pl.BlockSpec((1, NPG, 1), lambda b, j: (b, 0, 0)),
        ],
        out_shape=[jax.ShapeDtypeStruct((B, NPG, H), jnp.float32),
                   jax.ShapeDtypeStruct((B, NPG, H), jnp.float32),
                   jax.ShapeDtypeStruct((B, NPG, H), jnp.float32),
                   jax.ShapeDtypeStruct((B, NPG, H), jnp.float32),
                   jax.ShapeDtypeStruct((B, NPG, 1), jnp.float32)],
        scratch_shapes=[pltpu.VMEM((EB, H), jnp.float32),
                        pltpu.VMEM((EB, H), jnp.float32)],
    )(dstl_s, srcl_s, e_attr, xt3,
      lp['edge']['w'], lp['edge']['b'], w1, b1, w2, b2)
    return outs


def _post_body(mn_ref, mx_ref, sm_ref, sq_ref, ct_ref, xt_ref,
               wa_ref, wbcd_ref, pb_ref, lw_ref, lb_ref,
               out_ref, enz_ref):
    ct = ct_ref[0]
    c = jnp.maximum(ct, 1.0)
    mean = sm_ref[0] / c
    msq = sq_ref[0] / c
    std = jnp.sqrt(jnp.maximum(msq - mean * mean, 0.0) + 1e-5)
    mn = mn_ref[0]
    mx = mx_ref[0]
    amp = jnp.log(c + 1.0) / AVG_LOG
    lin_s = c / AVG_LIN
    xt = xt_ref[0]
    ys = []
    for t in range(T):
        sl = slice(t * F, (t + 1) * F)
        a = jnp.concatenate([mean[:, sl], mn[:, sl], mx[:, sl], std[:, sl]],
                            axis=1)
        z = jax.lax.dot_general(a, wbcd_ref[t], (((1,), (0,)), ((), ())),
                                preferred_element_type=jnp.float32)
        y = jax.lax.dot_general(xt[:, sl], wa_ref[t], (((1,), (0,)), ((), ())),
                                preferred_element_type=jnp.float32)
        y = y + z[:, :F] + amp * z[:, F:2 * F] + lin_s * z[:, 2 * F:] + pb_ref[t]
        ys.append(y)
    yy = jnp.concatenate(ys, axis=1)
    xnew = jax.lax.dot_general(yy, lw_ref[...], (((1,), (0,)), ((), ())),
                               preferred_element_type=jnp.float32) + lb_ref[...]
    out_ref[0] = xnew
    enz_ref[0] = jnp.max(xnew, axis=0, keepdims=True)


def _pna_post(mn, mx, sm, sq, ct, xt3, lp):
    wa = jnp.stack([lp['post'][t]['w'][:F] for t in range(T)])
    wbcd = jnp.stack([jnp.concatenate(
        [lp['post'][t]['w'][F:5 * F],
         lp['post'][t]['w'][5 * F:9 * F],
         lp['post'][t]['w'][9 * F:13 * F]], axis=1) for t in range(T)])
    pb = jnp.stack([lp['post'][t]['b'] for t in range(T)])
    return pl.pallas_call(
        _post_body,
        grid=(B,),
        in_specs=[
            pl.BlockSpec((1, NPG, H), lambda b: (b, 0, 0)),
            pl.BlockSpec((1, NPG, H), lambda b: (b, 0, 0)),
            pl.BlockSpec((1, NPG, H), lambda b: (b, 0, 0)),
            pl.BlockSpec((1, NPG, H), lambda b: (b, 0, 0)),
            pl.BlockSpec((1, NPG, 1), lambda b: (b, 0, 0)),
            pl.BlockSpec((1, NPG, H), lambda b: (b, 0, 0)),
            pl.BlockSpec((T, F, F), lambda b: (0, 0, 0)),
            pl.BlockSpec((T, 4 * F, 3 * F), lambda b: (0, 0, 0)),
            pl.BlockSpec((T, F), lambda b: (0, 0)),
            pl.BlockSpec((H, H), lambda b: (0, 0)),
            pl.BlockSpec((H,), lambda b: (0,)),
        ],
        out_specs=[
            pl.BlockSpec((1, NPG, H), lambda b: (b, 0, 0)),
            pl.BlockSpec((1, 1, H), lambda b: (b, 0, 0)),
        ],
        out_shape=[jax.ShapeDtypeStruct((B, NPG, H), jnp.float32),
                   jax.ShapeDtypeStruct((B, 1, H), jnp.float32)],
    )(mn, mx, sm, sq, ct, xt3, wa, wbcd, pb, lp['lin']['w'], lp['lin']['b'])


def _adj_body(dl_ref, sl_ref, adj_ref):
    j = pl.program_id(1)

    @pl.when(j == 0)
    def _():
        adj_ref[...] = jnp.zeros((1, NPG, NPG), jnp.float32)

    cols = jax.lax.broadcasted_iota(jnp.int32, (EB, NPG), 1)
    ms = (sl_ref[...] == cols).astype(jnp.float32)
    md = (dl_ref[...] == cols).astype(jnp.float32)
    adj_ref[0] = adj_ref[0] + jax.lax.dot_general(
        ms, md, (((0,), (0,)), ((), ())), preferred_element_type=jnp.float32)


def _build_adj(dstl_s, srcl_s):
    return pl.pallas_call(
        _adj_body,
        grid=(B, NBLK),
        in_specs=[
            pl.BlockSpec((EB, 1), lambda b, j: (b * NBLK + j, 0)),
            pl.BlockSpec((EB, 1), lambda b, j: (b * NBLK + j, 0)),
        ],
        out_specs=pl.BlockSpec((1, NPG, NPG), lambda b, j: (b, 0, 0)),
        out_shape=jax.ShapeDtypeStruct((B, NPG, NPG), jnp.float32),
    )(dstl_s, srcl_s)


def _gcn_pair_body(x_ref, adj_ref, dinv_ref, w1_ref, b1_ref, w2_ref, b2_ref,
                   s_ref):
    xg = x_ref[0]
    adj = adj_ref[0]
    dinv = dinv_ref[0]
    d2 = dinv * dinv
    xw1 = jax.lax.dot_general(xg, w1_ref[...], (((1,), (0,)), ((), ())),
                              preferred_element_type=jnp.float32)
    agg1 = dinv * jax.lax.dot_general(adj, dinv * xw1,
                                      (((0,), (0,)), ((), ())),
                                      preferred_element_type=jnp.float32)
    hsig = jax.nn.relu(agg1 + xw1 * d2 + b1_ref[...])
    xw2 = jax.lax.dot_general(hsig, w2_ref[...], (((1,), (0,)), ((), ())),
                              preferred_element_type=jnp.float32)
    agg2 = dinv * jax.lax.dot_general(adj, dinv * xw2,
                                      (((0,), (0,)), ((), ())),
                                      preferred_element_type=jnp.float32)
    s_ref[0] = agg2 + xw2 * d2 + b2_ref[...]


def _gcn_pair(x3, adj, dinv3, lp, K):
    return pl.pallas_call(
        _gcn_pair_body,
        grid=(B,),
        in_specs=[
            pl.BlockSpec((1, NPG, H), lambda b: (b, 0, 0)),
            pl.BlockSpec((1, NPG, NPG), lambda b: (b, 0, 0)),
            pl.BlockSpec((1, NPG, 1), lambda b: (b, 0, 0)),
            pl.BlockSpec((H, 2 * H), lambda b: (0, 0)),
            pl.BlockSpec((2 * H,), lambda b: (0,)),
            pl.BlockSpec((2 * H, K), lambda b: (0, 0)),
            pl.BlockSpec((K,), lambda b: (0,)),
        ],
        out_specs=pl.BlockSpec((1, NPG, K), lambda b: (b, 0, 0)),
        out_shape=jax.ShapeDtypeStruct((B, NPG, K), jnp.float32),
    )(x3, adj, dinv3, lp['gcn1']['w'], lp['gcn1']['b'],
      lp['gcn2']['w'], lp['gcn2']['b'])


def _apply(p, x):
    return x @ p['w'] + p['b']


def _rbf(d):
    d = jnp.minimum(d, 1.0)
    mu = jnp.linspace(0.0, 1.0, H)
    sig = 1.0 / H
    return jnp.exp(-(((d[:, None] - mu[None, :]) / sig) ** 2))


def _layer_norm(x, g, b):
    m = x.mean(-1, keepdims=True)
    v = ((x - m) ** 2).mean(-1, keepdims=True)
    return g * (x - m) / jnp.sqrt(v + 1e-5) + b


def _graph_norm(x, params):
    xr = x.reshape(B, NPG, H)
    mean = xr.mean(1, keepdims=True)
    out = xr - params['gn_ms'] * mean
    var = (out ** 2).mean(1, keepdims=True)
    out = out / jnp.sqrt(var + 1e-5)
    return (params['gn_w'] * out + params['gn_b']).reshape(N, H)


def _pna(p, x, dstl_s, srcl_s, e_attr):
    xt3 = x.reshape(B, NPG, H)
    mn, mx, sm, sq, ct = _seg_agg(dstl_s, srcl_s, e_attr, xt3, p)
    xnew, enz1 = _pna_post(mn, mx, sm, sq, ct, xt3, p)
    return xnew.reshape(N, H), enz1.reshape(B, H)


def _gcn(x, p, src, dst, dinv):
    xw = x @ p['w']
    nrm = (dinv[src] * dinv[dst])[:, None]
    out = jax.ops.segment_sum(xw[src] * nrm, dst, num_segments=N) \
        + xw * (dinv * dinv)[:, None]
    return out + p['b']


def _mincut(x, adj, s, K):
    s = jax.nn.softmax(s, -1)
    out = jnp.einsum('bnk,bnd->bkd', s, x)
    sa = jnp.einsum('bnk,bnm->bkm', s, adj)
    out_adj = jnp.einsum('bkm,bml->bkl', sa, s)
    num = jnp.trace(out_adj, axis1=1, axis2=2)
    d_flat = adj.sum(-1)
    den = (d_flat[..., None] * s * s).sum((1, 2))
    mincut_loss = -jnp.mean(num / (den + 1e-15))
    ss = jnp.einsum('bnk,bnl->bkl', s, s)
    i_s = jnp.eye(K, dtype=jnp.float32)
    ortho = jnp.linalg.norm(
        ss / jnp.linalg.norm(ss, axis=(-1, -2), keepdims=True)
        - i_s / jnp.linalg.norm(i_s), axis=(-1, -2))
    ortho_loss = jnp.mean(ortho)
    out_adj = out_adj * (1.0 - i_s)
    dd = jnp.sqrt(out_adj.sum(-1))[..., None] + 1e-15
    out_adj = out_adj / dd / jnp.transpose(dd, (0, 2, 1))
    return s, out, out_adj, mincut_loss, ortho_loss


def kernel(residue_x, residue_evo_x, residue_edge_index, residue_edge_weight,
           prot_batch, params):
    perm = jnp.argsort(residue_edge_index[1])
    src = residue_edge_index[0][perm]
    dst = residue_edge_index[1][perm]
    ew = residue_edge_weight[perm]
    dstl_s = (dst % NPG).astype(jnp.int32)[:, None]
    srcl_s = (src % NPG).astype(jnp.int32)[:, None]
    e_attr = _rbf(ew)
    x = _encoder(residue_x, residue_evo_x, params)
    adj = _build_adj(dstl_s, srcl_s)
    deg = adj.sum(1).reshape(N)
    dinv = 1.0 / jnp.sqrt(deg + 1.0)
    dinv3 = dinv.reshape(B, NPG, 1)
    ln = lambda z: _layer_norm(z, params['ln_g'], params['ln_b'])
    enz1 = []; enz2 = []; clus = []
    ortho_loss = jnp.asarray(0.0, jnp.float32)
    cluster_loss = jnp.asarray(0.0, jnp.float32)
    for idx, lp in enumerate(params['layers']):
        K = KS[idx]
        x = _graph_norm(x, params)
        x, e1v = _pna(lp, x, dstl_s, srcl_s, e_attr)
        enz1.append(e1v)
        s_d = _gcn_pair(x.reshape(B, NPG, H), adj, dinv3, lp, K)
        hx = x.reshape(B, NPG, H)
        s_sm, cx, _, cl, ol = _mincut(hx, adj, s_d, K)
        cx = ln(cx)
        ortho_loss = ortho_loss + ol
        cluster_loss = cluster_loss + cl
        clus.append(ln(cx.max(1)))
        x = x + jax.nn.relu(_apply(lp['res'],
                                   jnp.einsum('bnk,bkd->bnd', s_sm, cx).reshape(N, H)))
        enz2.append(ln(x.reshape(B, NPG, H).max(1)))
    e1 = jax.nn.relu(_apply(params['mol1'], jnp.concatenate(enz1, -1)))
    e2 = jax.nn.relu(_apply(params['mol2'], jnp.concatenate(enz2, -1)))
    cf = jax.nn.relu(_apply(params['mol3'], jnp.concatenate(clus, -1)))
    feat = jnp.concatenate([e1, e2, cf], -1)
    h0 = jax.nn.relu(_apply(params['cls0'], feat))
    h1 = jax.nn.relu(_apply(params['cls1'], h0))
    reg = _apply(params['cls2'], h1)
    return (reg, jnp.asarray(0.0, jnp.float32), ortho_loss, cluster_loss)
